# baseline (speedup 1.0000x reference)
import functools

import jax
import jax.numpy as jnp
from jax import lax
from jax.experimental import pallas as pl
from jax.experimental.pallas import tpu as pltpu
from jax.experimental.pallas import tpu_sc as plsc

B, S, D, V = 4, 4096, 768, 6
NC, NS, L = 2, 16, 16
NW = NC * NS
TOK = B * S
PER_W = TOK // NW
C = 64
NCHUNK = PER_W // C
NSLICE = D // L

_mesh = plsc.VectorSubcoreMesh(core_axis_name="c", subcore_axis_name="s")


@functools.partial(
    pl.kernel,
    out_type=jax.ShapeDtypeStruct((TOK, D), jnp.float32),
    mesh=_mesh,
    compiler_params=pltpu.CompilerParams(needs_layout_passes=False),
    scratch_types=[
        pltpu.VMEM((V, D), jnp.float32),
        pltpu.VMEM((D // 2,), jnp.int32),
        pltpu.VMEM((D // 2,), jnp.int32),
        pltpu.VMEM((C + L,), jnp.int32),
        pltpu.VMEM((C, D), jnp.float32),
        pltpu.VMEM((C, D), jnp.float32),
    ],
)
def _emb_ln(seqs_hbm, vocab_hbm, pos_hbm, gamma_hbm, beta_hbm, out_hbm,
            vocab_v, gamma_v, beta_v, idx_v, pos_v, x_v):
    cid = lax.axis_index("c")
    sid = lax.axis_index("s")
    wid = sid * NC + cid
    base = wid * PER_W
    s_base = (wid % (S // PER_W)) * PER_W

    pltpu.sync_copy(vocab_hbm, vocab_v)
    pltpu.sync_copy(gamma_hbm, gamma_v)
    pltpu.sync_copy(beta_hbm, beta_v)

    lanes = lax.iota(jnp.int32, L)

    def chunk_body(g, carry):
        t0 = base + g * C
        s0 = s_base + g * C
        pltpu.sync_copy(seqs_hbm.at[pl.ds(t0, C)], idx_v.at[pl.ds(0, C)])
        pltpu.sync_copy(pos_hbm.at[pl.ds(s0, C)], pos_v)

        @plsc.parallel_loop(0, C, step=1, unroll=4)
        def tok_body(i):
            row = idx_v[pl.ds(i, L)][0]
            acc_s = jnp.zeros((L,), jnp.float32)
            acc_q = jnp.zeros((L,), jnp.float32)
            for j in range(NSLICE):
                e = vocab_v[row, pl.ds(j * L, L)]
                p = pos_v[i, pl.ds(j * L, L)]
                x = e + p
                x_v[i, pl.ds(j * L, L)] = x
                acc_s = acc_s + x
                acc_q = acc_q + x * x
            mean = jnp.sum(acc_s) * (1.0 / D)
            var = jnp.sum(acc_q) * (1.0 / D) - mean * mean
            v16 = jnp.broadcast_to(var + 1e-12, (L,))
            yi = plsc.bitcast(v16, jnp.int32)
            yi = 0x5F3759DF - lax.shift_right_logical(yi, 1)
            y = plsc.bitcast(yi, jnp.float32)
            for _ in range(3):
                y = y * (1.5 - 0.5 * v16 * y * y)
            m16 = jnp.broadcast_to(mean, (L,))
            for j2 in range(NSLICE // 2):
                g2 = plsc.bitcast(gamma_v[pl.ds(j2 * L, L)], jnp.bfloat16)
                b2 = plsc.bitcast(beta_v[pl.ds(j2 * L, L)], jnp.bfloat16)
                gs = plsc.unpack(g2, format=plsc.PackFormat.INTERLEAVED,
                                 preferred_element_type=jnp.float32)
                bs = plsc.unpack(b2, format=plsc.PackFormat.INTERLEAVED,
                                 preferred_element_type=jnp.float32)
                for h in range(2):
                    j = j2 * 2 + h
                    x = x_v[i, pl.ds(j * L, L)]
                    x_v[i, pl.ds(j * L, L)] = (x - m16) * y * gs[h] + bs[h]

        pltpu.sync_copy(x_v, out_hbm.at[pl.ds(t0, C)])
        return carry

    lax.fori_loop(0, NCHUNK, chunk_body, 0)


def kernel(seqs, species, vocab_table, pos_table, gamma, beta):
    def _ileave(w):
        iv = (w.astype(jnp.bfloat16).reshape(D // 32, 2, L)
              .transpose(0, 2, 1).reshape(D // 2, 2))
        return jax.lax.bitcast_convert_type(iv, jnp.int32)

    out = _emb_ln(seqs.reshape(TOK), vocab_table, pos_table,
                  _ileave(gamma), _ileave(beta))
    return out.reshape(B, S, D)


# speedup vs baseline: 1.3565x; 1.3565x over previous
import functools

import jax
import jax.numpy as jnp
from jax import lax
from jax.experimental import pallas as pl
from jax.experimental.pallas import tpu as pltpu
from jax.experimental.pallas import tpu_sc as plsc

B, S, D, V = 4, 4096, 768, 6
NC, NS, L = 2, 16, 16
NW = NC * NS
TOK = B * S
PER_W = TOK // NW
C = 64
NCHUNK = PER_W // C
NSLICE = D // L

_mesh = plsc.VectorSubcoreMesh(core_axis_name="c", subcore_axis_name="s")


@functools.partial(
    pl.kernel,
    out_type=jax.ShapeDtypeStruct((TOK, D), jnp.float32),
    mesh=_mesh,
    compiler_params=pltpu.CompilerParams(needs_layout_passes=False),
    scratch_types=[
        pltpu.VMEM((V, D), jnp.float32),
        pltpu.VMEM((D // 2,), jnp.int32),
        pltpu.VMEM((D // 2,), jnp.int32),
        pltpu.VMEM((C + L,), jnp.int32),
        pltpu.VMEM((C, D), jnp.float32),
        pltpu.VMEM((C, D), jnp.float32),
    ],
)
def _emb_ln(seqs_hbm, vocab_hbm, pos_hbm, gamma_hbm, beta_hbm, out_hbm,
            vocab_v, gamma_v, beta_v, idx_v, pos_v, x_v):
    cid = lax.axis_index("c")
    sid = lax.axis_index("s")
    wid = sid * NC + cid
    base = wid * PER_W
    s_base = (wid % (S // PER_W)) * PER_W

    pltpu.sync_copy(vocab_hbm, vocab_v)
    pltpu.sync_copy(gamma_hbm, gamma_v)
    pltpu.sync_copy(beta_hbm, beta_v)

    lanes = lax.iota(jnp.int32, L)

    def chunk_body(g, carry):
        t0 = base + g * C
        s0 = s_base + g * C
        pltpu.sync_copy(seqs_hbm.at[pl.ds(t0, C)], idx_v.at[pl.ds(0, C)])
        pltpu.sync_copy(pos_hbm.at[pl.ds(s0, C)], pos_v)

        @plsc.parallel_loop(0, C, step=1, unroll=2)
        def tok_body(i):
            row = idx_v[pl.ds(i, L)][0]
            acc_s = jnp.zeros((L,), jnp.float32)
            acc_q = jnp.zeros((L,), jnp.float32)
            for j in range(NSLICE):
                e = vocab_v[row, pl.ds(j * L, L)]
                p = pos_v[i, pl.ds(j * L, L)]
                x = e + p
                x_v[i, pl.ds(j * L, L)] = x
                acc_s = acc_s + x
                acc_q = acc_q + x * x
            mean = jnp.sum(acc_s) * (1.0 / D)
            var = jnp.sum(acc_q) * (1.0 / D) - mean * mean
            v16 = jnp.broadcast_to(var + 1e-12, (L,))
            yi = plsc.bitcast(v16, jnp.int32)
            yi = 0x5F3759DF - lax.shift_right_logical(yi, 1)
            y = plsc.bitcast(yi, jnp.float32)
            for _ in range(3):
                y = y * (1.5 - 0.5 * v16 * y * y)
            m16 = jnp.broadcast_to(mean, (L,))
            for j2 in range(NSLICE // 2):
                g2 = plsc.bitcast(gamma_v[pl.ds(j2 * L, L)], jnp.bfloat16)
                b2 = plsc.bitcast(beta_v[pl.ds(j2 * L, L)], jnp.bfloat16)
                gs = plsc.unpack(g2, format=plsc.PackFormat.INTERLEAVED,
                                 preferred_element_type=jnp.float32)
                bs = plsc.unpack(b2, format=plsc.PackFormat.INTERLEAVED,
                                 preferred_element_type=jnp.float32)
                for h in range(2):
                    j = j2 * 2 + h
                    x = x_v[i, pl.ds(j * L, L)]
                    x_v[i, pl.ds(j * L, L)] = (x - m16) * y * gs[h] + bs[h]

        pltpu.sync_copy(x_v, out_hbm.at[pl.ds(t0, C)])
        return carry

    lax.fori_loop(0, NCHUNK, chunk_body, 0)


def kernel(seqs, species, vocab_table, pos_table, gamma, beta):
    def _ileave(w):
        iv = (w.astype(jnp.bfloat16).reshape(D // 32, 2, L)
              .transpose(0, 2, 1).reshape(D // 2, 2))
        return jax.lax.bitcast_convert_type(iv, jnp.int32)

    out = _emb_ln(seqs.reshape(TOK), vocab_table, pos_table,
                  _ileave(gamma), _ileave(beta))
    return out.reshape(B, S, D)
